# baseline (speedup 1.0000x reference)
import functools

import jax
from jax import lax
import jax.numpy as jnp
from jax.experimental import pallas as pl
from jax.experimental.pallas import tpu as pltpu
from jax.experimental.pallas import tpu_sc as plsc

SEQ = 4096
BATCH = 4
HIDDEN = 4096
ROWS = SEQ * BATCH

NW = 32
RPW = ROWS // NW
C = 8
NCH = RPW // C
NBUF = 3
NFULL = (NCH // NBUF) * NBUF

_vector_mesh = plsc.VectorSubcoreMesh(
    core_axis_name="core", subcore_axis_name="subcore"
)


@jax.jit
def _sc_gather(table, indices):
  @functools.partial(
      pl.kernel,
      out_type=jax.ShapeDtypeStruct((ROWS, HIDDEN), table.dtype),
      mesh=_vector_mesh,
      scratch_types=[
          pltpu.VMEM((RPW,), jnp.int32),
          *[pltpu.VMEM((C, HIDDEN), table.dtype) for _ in range(NBUF)],
          *[pltpu.SemaphoreType.DMA for _ in range(2 * NBUF)],
      ],
  )
  def kern(table_hbm, idx_hbm, out_hbm, idx_v, *scratch):
    bufs = scratch[:NBUF]
    gsems = scratch[NBUF:2 * NBUF]
    osems = scratch[2 * NBUF:]
    wid = lax.axis_index("subcore") * 2 + lax.axis_index("core")
    base = wid * RPW

    pltpu.sync_copy(idx_hbm.at[pl.ds(base, RPW)], idx_v)

    def start_gather(g, b):
      pltpu.async_copy(
          table_hbm.at[idx_v.at[pl.ds(g * C, C)]], bufs[b], gsems[b]
      )

    def wait_gather(b):
      pltpu.make_async_copy(table_hbm.at[pl.ds(0, C)], bufs[b],
                            gsems[b]).wait()

    def wait_out(b):
      pltpu.make_async_copy(bufs[b], out_hbm.at[pl.ds(0, C)],
                            osems[b]).wait()

    for b in range(NBUF):
      start_gather(b, b)

    @pl.loop(0, NFULL, step=NBUF)
    def _(c0):
      for b in range(NBUF):
        g = c0 + b
        wait_gather(b)
        pltpu.async_copy(bufs[b], out_hbm.at[pl.ds(base + g * C, C)],
                         osems[b])
        wait_out(b)

        @pl.when(g + NBUF < NCH)
        def _():
          start_gather(g + NBUF, b)

    for g in range(NFULL, NCH):
      b = g % NBUF
      wait_gather(b)
      pltpu.async_copy(bufs[b], out_hbm.at[pl.ds(base + g * C, C)],
                       osems[b])
      wait_out(b)

  return kern(table, indices)


def kernel(position_ids, embedding_table):
  idx = jnp.transpose(position_ids).reshape(ROWS).astype(jnp.int32)
  out = _sc_gather(embedding_table, idx)
  return out.reshape(BATCH, SEQ, HIDDEN)


# speedup vs baseline: 1.0278x; 1.0278x over previous
import dataclasses
import functools

import jax
from jax import lax
import jax.numpy as jnp
from jax.experimental import pallas as pl
from jax.experimental.pallas import tpu as pltpu
from jax.experimental.pallas import tpu_sc as plsc

SEQ = 4096
BATCH = 4
HIDDEN = 4096
ROWS = SEQ * BATCH
MAXPOS = 8192

NW = 32
BKT = MAXPOS // NW
SB = 8
NS = BKT // SB
NRB = 2
SMC = 512
NOUT = 8

_vector_mesh = plsc.VectorSubcoreMesh(
    core_axis_name="core", subcore_axis_name="subcore"
)

_cp = pltpu.CompilerParams()
if "needs_layout_passes" in pltpu.CompilerParams.__dataclass_fields__:
  _cp = dataclasses.replace(_cp, needs_layout_passes=False)


@jax.jit
def _sc_scatter_gather(table, indices):
  @functools.partial(
      pl.kernel,
      out_type=jax.ShapeDtypeStruct((ROWS, HIDDEN), table.dtype),
      mesh=_vector_mesh,
      compiler_params=_cp,
      scratch_types=[
          pltpu.VMEM((ROWS,), jnp.int32),
          pltpu.VMEM((ROWS,), jnp.int32),
          pltpu.VMEM((ROWS,), jnp.int32),
          *[pltpu.VMEM((SB, HIDDEN), table.dtype) for _ in range(NRB)],
          *[pltpu.SemaphoreType.DMA for _ in range(NRB)],
          pltpu.SemaphoreType.DMA,
      ],
  )
  def kern(table_hbm, idx_hbm, out_hbm, idxs, bkt_buf, sub_buf,
           *scratch):
    rowbufs = scratch[:NRB]
    lsems = scratch[NRB:2 * NRB]
    osem = scratch[2 * NRB]
    wid = lax.axis_index("subcore") * 2 + lax.axis_index("core")
    r0 = wid * BKT

    def start_load(s, b):
      pltpu.async_copy(table_hbm.at[pl.ds(r0 + s * SB, SB)], rowbufs[b],
                       lsems[b])

    def wait_load(b):
      pltpu.make_async_copy(table_hbm.at[pl.ds(0, SB)], rowbufs[b],
                            lsems[b]).wait()

    for b in range(NRB):
      start_load(b, b)

    pltpu.sync_copy(idx_hbm, idxs)

    iota = lax.iota(jnp.int32, 16)
    ones = jnp.ones((16,), jnp.int32)
    zero16 = jnp.zeros((16,), jnp.int32)

    def p1_body(i, off):
      v = idxs[pl.ds(i * 16, 16)]
      local = v - r0
      m = (local >= 0) & (local < BKT)
      packed = (iota + i * 16) * 256 + local
      dst = off + plsc.cumsum(jnp.where(m, ones, zero16)) - 1
      plsc.store_scatter(bkt_buf, [dst], packed, mask=m)
      return off + plsc.all_reduce_population_count(m)

    off = lax.fori_loop(0, ROWS // 16, p1_body, zero16)
    total = jnp.max(off)
    nvec = (total + 15) // 16

    for s in range(NS):
      bsel = s % NRB

      def p2_body(j, off2, s=s):
        v = bkt_buf[pl.ds(j * 16, 16)]
        local = v & 255
        m = ((local >= s * SB) & (local < (s + 1) * SB)
             & (iota + j * 16 < total))
        packed = (v >> 8) * 8 + (local & 7)
        dst = off2 + plsc.cumsum(jnp.where(m, ones, zero16)) - 1
        plsc.store_scatter(sub_buf, [dst], packed, mask=m)
        return off2 + plsc.all_reduce_population_count(m)

      k2 = jnp.max(lax.fori_loop(0, nvec, p2_body, zero16))

      wait_load(bsel)

      def wr_body(j, carry, bsel=bsel):
        v = sub_buf[pl.ds((j // 16) * 16, 16)]
        w = jnp.sum(jnp.where(iota == lax.rem(j, 16), v, zero16))
        pltpu.async_copy(rowbufs[bsel].at[pl.ds(w & 7, 1)],
                         out_hbm.at[pl.ds(w >> 3, 1)], osem)

        @pl.when(j >= NOUT)
        def _():
          pltpu.make_async_copy(rowbufs[bsel].at[pl.ds(0, 1)],
                                out_hbm.at[pl.ds(0, 1)], osem).wait()

        return carry

      lax.fori_loop(0, k2, wr_body, 0)

      def drain_body(j, carry, bsel=bsel):
        pltpu.make_async_copy(rowbufs[bsel].at[pl.ds(0, 1)],
                              out_hbm.at[pl.ds(0, 1)], osem).wait()
        return carry

      lax.fori_loop(0, jnp.minimum(k2, NOUT), drain_body, 0)

      if s + NRB < NS:
        start_load(s + NRB, bsel)

  return kern(table, indices)


def kernel(position_ids, embedding_table):
  idx = jnp.transpose(position_ids).reshape(ROWS).astype(jnp.int32)
  out = _sc_scatter_gather(embedding_table, idx)
  return out.reshape(BATCH, SEQ, HIDDEN)


# speedup vs baseline: 1.0598x; 1.0312x over previous
import dataclasses
import functools

import jax
from jax import lax
import jax.numpy as jnp
from jax.experimental import pallas as pl
from jax.experimental.pallas import tpu as pltpu
from jax.experimental.pallas import tpu_sc as plsc

SEQ = 4096
BATCH = 4
HIDDEN = 4096
ROWS = SEQ * BATCH
MAXPOS = 8192

NW = 32
BKT = MAXPOS // NW
SB = 8
NS = BKT // SB
NRB = 2
NOUT = 8

_vector_mesh = plsc.VectorSubcoreMesh(
    core_axis_name="core", subcore_axis_name="subcore"
)

_cp = pltpu.CompilerParams()
if "needs_layout_passes" in pltpu.CompilerParams.__dataclass_fields__:
  _cp = dataclasses.replace(_cp, needs_layout_passes=False)


@jax.jit
def _sc_scatter_gather(table, indices):
  @functools.partial(
      pl.kernel,
      out_type=jax.ShapeDtypeStruct((ROWS, HIDDEN), table.dtype),
      mesh=_vector_mesh,
      compiler_params=_cp,
      scratch_types=[
          pltpu.VMEM((ROWS,), jnp.int32),
          pltpu.VMEM((ROWS,), jnp.int32),
          pltpu.VMEM((ROWS,), jnp.int32),
          *[pltpu.VMEM((SB, HIDDEN), table.dtype) for _ in range(NRB)],
          *[pltpu.SemaphoreType.DMA for _ in range(NRB)],
          pltpu.SemaphoreType.DMA,
      ],
  )
  def kern(table_hbm, idx_hbm, out_hbm, idxs, bkt_buf, sub_buf,
           *scratch):
    rowbufs = scratch[:NRB]
    lsems = scratch[NRB:2 * NRB]
    osem = scratch[2 * NRB]
    wid = lax.axis_index("subcore") * 2 + lax.axis_index("core")
    r0 = wid * BKT

    def start_load(s, b):
      pltpu.async_copy(table_hbm.at[pl.ds(r0 + s * SB, SB)], rowbufs[b],
                       lsems[b])

    def wait_load(b):
      pltpu.make_async_copy(table_hbm.at[pl.ds(0, SB)], rowbufs[b],
                            lsems[b]).wait()

    def wait_one_write(b):
      pltpu.make_async_copy(rowbufs[b].at[pl.ds(0, 1)],
                            out_hbm.at[pl.ds(0, 1)], osem).wait()

    for b in range(NRB):
      start_load(b, b)

    pltpu.sync_copy(idx_hbm, idxs)

    iota = lax.iota(jnp.int32, 16)
    ones = jnp.ones((16,), jnp.int32)
    zero16 = jnp.zeros((16,), jnp.int32)

    def p1_body(i, off):
      v = idxs[pl.ds(i * 16, 16)]
      local = v - r0
      m = (local >= 0) & (local < BKT)
      packed = (iota + i * 16) * 256 + local
      dst = off + plsc.cumsum(jnp.where(m, ones, zero16)) - 1
      plsc.store_scatter(bkt_buf, [dst], packed, mask=m)
      return off + plsc.all_reduce_population_count(m)

    off = lax.fori_loop(0, ROWS // 16, p1_body, zero16)
    total = jnp.max(off)
    nvec = (total + 15) // 16

    @pl.loop(0, NS, step=NRB)
    def _(s0):
      for b in range(NRB):
        s = s0 + b

        def p2_body(j, off2, s=s):
          v = bkt_buf[pl.ds(j * 16, 16)]
          local = v & 255
          m = ((local >= s * SB) & (local < (s + 1) * SB)
               & (iota + j * 16 < total))
          packed = (v >> 8) * 8 + (local & 7)
          dst = off2 + plsc.cumsum(jnp.where(m, ones, zero16)) - 1
          plsc.store_scatter(sub_buf, [dst], packed, mask=m)
          return off2 + plsc.all_reduce_population_count(m)

        k2 = jnp.max(lax.fori_loop(0, nvec, p2_body, zero16))

        wait_load(b)

        def wr_body(jv, carry, b=b, k2=k2):
          v = sub_buf[pl.ds(jv * 16, 16)]
          for l in range(16):
            j = jv * 16 + l
            valid = j < k2

            @pl.when(valid)
            def _(l=l, b=b):
              w = v[l]
              pltpu.async_copy(rowbufs[b].at[pl.ds(w & 7, 1)],
                               out_hbm.at[pl.ds(w >> 3, 1)], osem)

            @pl.when(valid & (j >= NOUT))
            def _(b=b):
              wait_one_write(b)

          return carry

        lax.fori_loop(0, (k2 + 15) // 16, wr_body, 0)

        def drain_body(j, carry, b=b):
          wait_one_write(b)
          return carry

        lax.fori_loop(0, jnp.minimum(k2, NOUT), drain_body, 0)

        @pl.when(s + NRB < NS)
        def _(s=s, b=b):
          start_load(s + NRB, b)

  return kern(table, indices)


def kernel(position_ids, embedding_table):
  idx = jnp.transpose(position_ids).reshape(ROWS).astype(jnp.int32)
  out = _sc_scatter_gather(embedding_table, idx)
  return out.reshape(BATCH, SEQ, HIDDEN)


# speedup vs baseline: 1.0643x; 1.0042x over previous
import dataclasses
import functools

import jax
from jax import lax
import jax.numpy as jnp
from jax.experimental import pallas as pl
from jax.experimental.pallas import tpu as pltpu
from jax.experimental.pallas import tpu_sc as plsc

SEQ = 4096
BATCH = 4
HIDDEN = 4096
ROWS = SEQ * BATCH
MAXPOS = 8192

NW = 32
BKT = MAXPOS // NW
SB = 8
NS = BKT // SB
NRB = 2
NOUT = 8

_vector_mesh = plsc.VectorSubcoreMesh(
    core_axis_name="core", subcore_axis_name="subcore"
)

_cp = pltpu.CompilerParams()
if "needs_layout_passes" in pltpu.CompilerParams.__dataclass_fields__:
  _cp = dataclasses.replace(_cp, needs_layout_passes=False)


@jax.jit
def _sc_scatter_gather(table, indices):
  @functools.partial(
      pl.kernel,
      out_type=jax.ShapeDtypeStruct((ROWS, HIDDEN), table.dtype),
      mesh=_vector_mesh,
      compiler_params=_cp,
      scratch_types=[
          pltpu.VMEM((ROWS,), jnp.int32),
          pltpu.VMEM((ROWS,), jnp.int32),
          pltpu.VMEM((ROWS,), jnp.int32),
          *[pltpu.VMEM((SB, HIDDEN), table.dtype) for _ in range(NRB)],
          *[pltpu.SemaphoreType.DMA for _ in range(NRB)],
          *[pltpu.SemaphoreType.DMA for _ in range(NRB)],
      ],
  )
  def kern(table_hbm, idx_hbm, out_hbm, idxs, bkt_buf, sub_buf,
           *scratch):
    rowbufs = scratch[:NRB]
    lsems = scratch[NRB:2 * NRB]
    osems = scratch[2 * NRB:]
    wid = lax.axis_index("subcore") * 2 + lax.axis_index("core")
    r0 = wid * BKT

    def start_load(s, b):
      pltpu.async_copy(table_hbm.at[pl.ds(r0 + s * SB, SB)], rowbufs[b],
                       lsems[b])

    def wait_load(b):
      pltpu.make_async_copy(table_hbm.at[pl.ds(0, SB)], rowbufs[b],
                            lsems[b]).wait()

    def wait_one_write(b):
      pltpu.make_async_copy(rowbufs[b].at[pl.ds(0, 1)],
                            out_hbm.at[pl.ds(0, 1)], osems[b]).wait()

    start_load(0, 0)

    pltpu.sync_copy(idx_hbm, idxs)

    iota = lax.iota(jnp.int32, 16)
    ones = jnp.ones((16,), jnp.int32)
    zero16 = jnp.zeros((16,), jnp.int32)

    def p1_body(i, off):
      v = idxs[pl.ds(i * 16, 16)]
      local = v - r0
      m = (local >= 0) & (local < BKT)
      packed = (iota + i * 16) * 256 + local
      dst = off + plsc.cumsum(jnp.where(m, ones, zero16)) - 1
      plsc.store_scatter(bkt_buf, [dst], packed, mask=m)
      return off + plsc.all_reduce_population_count(m)

    off = lax.fori_loop(0, ROWS // 16, p1_body, zero16)
    total = jnp.max(off)
    nvec = (total + 15) // 16

    def s_pair_body(t, kprev):
      s0 = t * NRB
      for b in range(NRB):
        s = s0 + b
        bn = (b + 1) % NRB

        def p2_body(j, off2, s=s):
          v = bkt_buf[pl.ds(j * 16, 16)]
          local = v & 255
          m = ((local >= s * SB) & (local < (s + 1) * SB)
               & (iota + j * 16 < total))
          packed = (v >> 8) * 8 + (local & 7)
          dst = off2 + plsc.cumsum(jnp.where(m, ones, zero16)) - 1
          plsc.store_scatter(sub_buf, [dst], packed, mask=m)
          return off2 + plsc.all_reduce_population_count(m)

        k2 = jnp.max(lax.fori_loop(0, nvec, p2_body, zero16))

        def late_drain_body(j, carry, bn=bn):
          wait_one_write(bn)
          return carry

        lax.fori_loop(0, jnp.minimum(kprev[bn], NOUT), late_drain_body, 0)
        kprev = tuple(jnp.int32(0) if i == bn else kprev[i]
                      for i in range(NRB))

        @pl.when(s + 1 < NS)
        def _(s=s, bn=bn):
          start_load(s + 1, bn)

        wait_load(b)

        def wr_body(jv, carry, b=b, k2=k2):
          v = sub_buf[pl.ds(jv * 16, 16)]
          for l in range(16):
            j = jv * 16 + l
            valid = j < k2

            @pl.when(valid)
            def _(l=l, b=b):
              w = v[l]
              pltpu.async_copy(rowbufs[b].at[pl.ds(w & 7, 1)],
                               out_hbm.at[pl.ds(w >> 3, 1)], osems[b])

            @pl.when(valid & (j >= NOUT))
            def _(b=b):
              wait_one_write(b)

          return carry

        lax.fori_loop(0, (k2 + 15) // 16, wr_body, 0)
        kprev = tuple(k2 if i == b else kprev[i] for i in range(NRB))
      return kprev

    kfin = lax.fori_loop(0, NS // NRB, s_pair_body,
                         tuple(jnp.int32(0) for _ in range(NRB)))

    for b in range(NRB):

      def fin_drain_body(j, carry, b=b):
        wait_one_write(b)
        return carry

      lax.fori_loop(0, jnp.minimum(kfin[b], NOUT), fin_drain_body, 0)

  return kern(table, indices)


def kernel(position_ids, embedding_table):
  idx = jnp.transpose(position_ids).reshape(ROWS).astype(jnp.int32)
  out = _sc_scatter_gather(embedding_table, idx)
  return out.reshape(BATCH, SEQ, HIDDEN)


# speedup vs baseline: 1.0652x; 1.0008x over previous
import dataclasses
import functools

import jax
from jax import lax
import jax.numpy as jnp
from jax.experimental import pallas as pl
from jax.experimental.pallas import tpu as pltpu
from jax.experimental.pallas import tpu_sc as plsc

SEQ = 4096
BATCH = 4
HIDDEN = 4096
ROWS = SEQ * BATCH
MAXPOS = 8192

NW = 32
BKT = MAXPOS // NW
SB = 8
NS = BKT // SB
NRB = 2
NOUT = 16

_vector_mesh = plsc.VectorSubcoreMesh(
    core_axis_name="core", subcore_axis_name="subcore"
)

_cp = pltpu.CompilerParams()
if "needs_layout_passes" in pltpu.CompilerParams.__dataclass_fields__:
  _cp = dataclasses.replace(_cp, needs_layout_passes=False)


@jax.jit
def _sc_scatter_gather(table, indices):
  @functools.partial(
      pl.kernel,
      out_type=jax.ShapeDtypeStruct((ROWS, HIDDEN), table.dtype),
      mesh=_vector_mesh,
      compiler_params=_cp,
      scratch_types=[
          pltpu.VMEM((ROWS,), jnp.int32),
          pltpu.VMEM((ROWS,), jnp.int32),
          pltpu.VMEM((ROWS,), jnp.int32),
          *[pltpu.VMEM((SB, HIDDEN), table.dtype) for _ in range(NRB)],
          *[pltpu.SemaphoreType.DMA for _ in range(NRB)],
          *[pltpu.SemaphoreType.DMA for _ in range(NRB)],
      ],
  )
  def kern(table_hbm, idx_hbm, out_hbm, idxs, bkt_buf, sub_buf,
           *scratch):
    rowbufs = scratch[:NRB]
    lsems = scratch[NRB:2 * NRB]
    osems = scratch[2 * NRB:]
    wid = lax.axis_index("subcore") * 2 + lax.axis_index("core")
    r0 = wid * BKT

    def start_load(s, b):
      pltpu.async_copy(table_hbm.at[pl.ds(r0 + s * SB, SB)], rowbufs[b],
                       lsems[b])

    def wait_load(b):
      pltpu.make_async_copy(table_hbm.at[pl.ds(0, SB)], rowbufs[b],
                            lsems[b]).wait()

    def wait_one_write(b):
      pltpu.make_async_copy(rowbufs[b].at[pl.ds(0, 1)],
                            out_hbm.at[pl.ds(0, 1)], osems[b]).wait()

    start_load(0, 0)

    pltpu.sync_copy(idx_hbm, idxs)

    iota = lax.iota(jnp.int32, 16)
    ones = jnp.ones((16,), jnp.int32)
    zero16 = jnp.zeros((16,), jnp.int32)

    def p1_body(i, off):
      v = idxs[pl.ds(i * 16, 16)]
      local = v - r0
      m = (local >= 0) & (local < BKT)
      packed = (iota + i * 16) * 256 + local
      dst = off + plsc.cumsum(jnp.where(m, ones, zero16)) - 1
      plsc.store_scatter(bkt_buf, [dst], packed, mask=m)
      return off + plsc.all_reduce_population_count(m)

    off = lax.fori_loop(0, ROWS // 16, p1_body, zero16)
    total = jnp.max(off)
    nvec = (total + 15) // 16

    def s_pair_body(t, kprev):
      s0 = t * NRB
      for b in range(NRB):
        s = s0 + b
        bn = (b + 1) % NRB

        def p2_body(j, off2, s=s):
          v = bkt_buf[pl.ds(j * 16, 16)]
          local = v & 255
          m = ((local >= s * SB) & (local < (s + 1) * SB)
               & (iota + j * 16 < total))
          packed = (v >> 8) * 8 + (local & 7)
          dst = off2 + plsc.cumsum(jnp.where(m, ones, zero16)) - 1
          plsc.store_scatter(sub_buf, [dst], packed, mask=m)
          return off2 + plsc.all_reduce_population_count(m)

        k2 = jnp.max(lax.fori_loop(0, nvec, p2_body, zero16))

        def late_drain_body(j, carry, bn=bn):
          wait_one_write(bn)
          return carry

        lax.fori_loop(0, jnp.minimum(kprev[bn], NOUT), late_drain_body, 0)
        kprev = tuple(jnp.int32(0) if i == bn else kprev[i]
                      for i in range(NRB))

        @pl.when(s + 1 < NS)
        def _(s=s, bn=bn):
          start_load(s + 1, bn)

        wait_load(b)

        def wr_body(jv, carry, b=b, k2=k2):
          v = sub_buf[pl.ds(jv * 16, 16)]
          for l in range(16):
            j = jv * 16 + l
            valid = j < k2

            @pl.when(valid)
            def _(l=l, b=b):
              w = v[l]
              pltpu.async_copy(rowbufs[b].at[pl.ds(w & 7, 1)],
                               out_hbm.at[pl.ds(w >> 3, 1)], osems[b])

            @pl.when(valid & (j >= NOUT))
            def _(b=b):
              wait_one_write(b)

          return carry

        lax.fori_loop(0, (k2 + 15) // 16, wr_body, 0)
        kprev = tuple(k2 if i == b else kprev[i] for i in range(NRB))
      return kprev

    kfin = lax.fori_loop(0, NS // NRB, s_pair_body,
                         tuple(jnp.int32(0) for _ in range(NRB)))

    for b in range(NRB):

      def fin_drain_body(j, carry, b=b):
        wait_one_write(b)
        return carry

      lax.fori_loop(0, jnp.minimum(kfin[b], NOUT), fin_drain_body, 0)

  return kern(table, indices)


def kernel(position_ids, embedding_table):
  idx = jnp.transpose(position_ids).reshape(ROWS).astype(jnp.int32)
  out = _sc_scatter_gather(embedding_table, idx)
  return out.reshape(BATCH, SEQ, HIDDEN)


# speedup vs baseline: 1.0664x; 1.0012x over previous
import dataclasses
import functools

import jax
from jax import lax
import jax.numpy as jnp
from jax.experimental import pallas as pl
from jax.experimental.pallas import tpu as pltpu
from jax.experimental.pallas import tpu_sc as plsc

SEQ = 4096
BATCH = 4
HIDDEN = 4096
ROWS = SEQ * BATCH
MAXPOS = 8192

NW = 32
BKT = MAXPOS // NW
SB = 8
NS = BKT // SB
NRB = 2
NOUT = 16

_vector_mesh = plsc.VectorSubcoreMesh(
    core_axis_name="core", subcore_axis_name="subcore"
)

_cp = pltpu.CompilerParams()
if "needs_layout_passes" in pltpu.CompilerParams.__dataclass_fields__:
  _cp = dataclasses.replace(_cp, needs_layout_passes=False)


@jax.jit
def _sc_scatter_gather(table, indices):
  @functools.partial(
      pl.kernel,
      out_type=jax.ShapeDtypeStruct((ROWS, HIDDEN), table.dtype),
      mesh=_vector_mesh,
      compiler_params=_cp,
      scratch_types=[
          pltpu.VMEM((ROWS,), jnp.int32),
          pltpu.VMEM((ROWS,), jnp.int32),
          pltpu.VMEM((ROWS,), jnp.int32),
          *[pltpu.VMEM((SB, HIDDEN), table.dtype) for _ in range(NRB)],
          *[pltpu.SemaphoreType.DMA for _ in range(NRB)],
          *[pltpu.SemaphoreType.DMA for _ in range(NRB)],
      ],
  )
  def kern(table_hbm, idx_hbm, out_hbm, idxs, bkt_buf, sub_buf,
           *scratch):
    rowbufs = scratch[:NRB]
    lsems = scratch[NRB:2 * NRB]
    osems = scratch[2 * NRB:]
    wid = lax.axis_index("subcore") * 2 + lax.axis_index("core")
    r0 = wid * BKT

    def start_load(s, b):
      pltpu.async_copy(table_hbm.at[pl.ds(r0 + s * SB, SB)], rowbufs[b],
                       lsems[b])

    def wait_load(b):
      pltpu.make_async_copy(table_hbm.at[pl.ds(0, SB)], rowbufs[b],
                            lsems[b]).wait()

    def wait_one_write(b):
      pltpu.make_async_copy(rowbufs[b].at[pl.ds(0, 1)],
                            out_hbm.at[pl.ds(0, 1)], osems[b]).wait()

    start_load(0, 0)

    pltpu.sync_copy(idx_hbm, idxs)

    iota = lax.iota(jnp.int32, 16)
    ones = jnp.ones((16,), jnp.int32)
    zero16 = jnp.zeros((16,), jnp.int32)

    def p1_body(i, off):
      for u in range(2):
        v = idxs[pl.ds(i * 32 + u * 16, 16)]
        local = v - r0
        m = (local >= 0) & (local < BKT)
        packed = (iota + (i * 32 + u * 16)) * 256 + local
        dst = off + plsc.cumsum(jnp.where(m, ones, zero16)) - 1
        plsc.store_scatter(bkt_buf, [dst], packed, mask=m)
        off = off + plsc.all_reduce_population_count(m)
      return off

    off = lax.fori_loop(0, ROWS // 32, p1_body, zero16)
    total = jnp.max(off)
    nvec = (total + 15) // 16

    def s_pair_body(t, kprev):
      s0 = t * NRB
      for b in range(NRB):
        s = s0 + b
        bn = (b + 1) % NRB

        def p2_body(j, off2, s=s):
          v = bkt_buf[pl.ds(j * 16, 16)]
          local = v & 255
          m = ((local >= s * SB) & (local < (s + 1) * SB)
               & (iota + j * 16 < total))
          packed = (v >> 8) * 8 + (local & 7)
          dst = off2 + plsc.cumsum(jnp.where(m, ones, zero16)) - 1
          plsc.store_scatter(sub_buf, [dst], packed, mask=m)
          return off2 + plsc.all_reduce_population_count(m)

        k2 = jnp.max(lax.fori_loop(0, nvec, p2_body, zero16))

        def late_drain_body(j, carry, bn=bn):
          wait_one_write(bn)
          return carry

        lax.fori_loop(0, jnp.minimum(kprev[bn], NOUT), late_drain_body, 0)
        kprev = tuple(jnp.int32(0) if i == bn else kprev[i]
                      for i in range(NRB))

        @pl.when(s + 1 < NS)
        def _(s=s, bn=bn):
          start_load(s + 1, bn)

        wait_load(b)

        def wr_body(jv, carry, b=b, k2=k2):
          v = sub_buf[pl.ds(jv * 16, 16)]
          for l in range(16):
            j = jv * 16 + l
            valid = j < k2

            @pl.when(valid)
            def _(l=l, b=b):
              w = v[l]
              pltpu.async_copy(rowbufs[b].at[pl.ds(w & 7, 1)],
                               out_hbm.at[pl.ds(w >> 3, 1)], osems[b])

            @pl.when(valid & (j >= NOUT))
            def _(b=b):
              wait_one_write(b)

          return carry

        lax.fori_loop(0, (k2 + 15) // 16, wr_body, 0)
        kprev = tuple(k2 if i == b else kprev[i] for i in range(NRB))
      return kprev

    kfin = lax.fori_loop(0, NS // NRB, s_pair_body,
                         tuple(jnp.int32(0) for _ in range(NRB)))

    for b in range(NRB):

      def fin_drain_body(j, carry, b=b):
        wait_one_write(b)
        return carry

      lax.fori_loop(0, jnp.minimum(kfin[b], NOUT), fin_drain_body, 0)

  return kern(table, indices)


def kernel(position_ids, embedding_table):
  idx = jnp.transpose(position_ids).reshape(ROWS).astype(jnp.int32)
  out = _sc_scatter_gather(embedding_table, idx)
  return out.reshape(BATCH, SEQ, HIDDEN)
